# baseline (speedup 1.0000x reference)
import jax
import jax.numpy as jnp
from jax import lax
from jax.experimental import pallas as pl
from jax.experimental.pallas import tpu as pltpu
from jax.experimental.pallas import tpu_sc as plsc

UN = 16
FEAT = 512
BATCH = 16
EN = UN * UN * UN
GROUPS = BATCH * UN * UN
LANES = 16
NWORK = 32
GPW = GROUPS // NWORK
W = 2
STEPS = GPW // W
NBUF = 8
REP = 1


def _uncap_body(x_hbm, mask_hbm, out_hbm, mvec, inbuf, outbuf, *sems):
    rsem = sems[:NBUF]
    wsem = sems[NBUF:]
    nc = 2
    cid = lax.axis_index("c")
    sid = lax.axis_index("s")
    wid = sid * nc + cid
    gbase = wid * GPW
    b = gbase // (UN * UN)
    m0 = gbase - b * (UN * UN)
    srcbase = b * EN + m0

    pltpu.sync_copy(mask_hbm.at[b], mvec)

    def sum_body(i, acc):
        return acc + mvec[pl.ds(i * LANES, LANES)]

    acc = lax.fori_loop(0, EN // LANES, sum_body, jnp.zeros((LANES,), jnp.int32))
    s = jnp.int32(0)
    for i in range(LANES):
        s = s + acc[i]
    s = s.astype(jnp.float32)
    ull = jnp.int32(0)
    for u in range(1, LANES + 1):
        ull = ull + (s >= jnp.float32((u - 0.5) ** 3)).astype(jnp.int32)

    zerov = jnp.zeros((LANES,), jnp.float32)

    def read(j, k):
        return pltpu.make_async_copy(
            x_hbm.at[pl.ds(srcbase + j * W, W)], inbuf.at[k], rsem[k]
        )

    def write(j, k, w, t):
        g = gbase + j * W + w
        return pltpu.make_async_copy(
            outbuf.at[k, w], out_hbm.at[pl.ds(g * UN + t * REP, REP)], wsem[k]
        )

    for k in range(NBUF):
        read(k, k).start()

    def outer(jj, carry):
        for k in range(NBUF):
            j = jj * NBUF + k
            read(j, k).wait()

            @pl.when(jj > 0)
            def _wait_prev_write():
                for w in range(W):
                    for t in range(UN // REP):
                        write(j - NBUF, k, w, t).wait()

            for w in range(W):
                m = m0 + j * W + w
                i0 = m // UN
                i1 = m - i0 * UN
                valid = (i0 < ull) & (i1 < ull)

                @pl.when(valid)
                def _rep():
                    def rep_body(c, _):
                        c16 = c * LANES
                        v = inbuf[k, w, pl.ds(c16, LANES)]
                        for rep in range(REP):
                            outbuf[k, w, rep, pl.ds(c16, LANES)] = v
                        return 0

                    lax.fori_loop(0, FEAT // LANES, rep_body, 0, unroll=4)

                @pl.when(jnp.logical_not(valid))
                def _zero():
                    def zero_body(c, _):
                        c16 = c * LANES
                        for rep in range(REP):
                            outbuf[k, w, rep, pl.ds(c16, LANES)] = zerov
                        return 0

                    lax.fori_loop(0, FEAT // LANES, zero_body, 0)

                for t in range(UN // REP):
                    write(j, k, w, t).start()

            @pl.when(j + NBUF < STEPS)
            def _next_read():
                read(j + NBUF, k).start()

        return carry

    lax.fori_loop(0, STEPS // NBUF, outer, 0)

    for k in range(NBUF):
        for w in range(W):
            for t in range(UN // REP):
                write(STEPS - NBUF + k, k, w, t).wait()


def kernel(x, mask):
    x2 = x.reshape(BATCH * EN, FEAT)
    mesh = plsc.VectorSubcoreMesh(core_axis_name="c", subcore_axis_name="s")
    run = pl.kernel(
        _uncap_body,
        out_type=jax.ShapeDtypeStruct((GROUPS * UN, FEAT), jnp.float32),
        mesh=mesh,
        scratch_types=(
            [
                pltpu.VMEM((EN,), jnp.int32),
                pltpu.VMEM((NBUF, W, FEAT), jnp.float32),
                pltpu.VMEM((NBUF, W, REP, FEAT), jnp.float32),
            ]
            + [pltpu.SemaphoreType.DMA] * (2 * NBUF)
        ),
    )
    out = run(x2, mask)
    return out.reshape(BATCH, UN, UN, UN, FEAT)


# speedup vs baseline: 1.0361x; 1.0361x over previous
import jax
import jax.numpy as jnp
from jax import lax
from jax.experimental import pallas as pl
from jax.experimental.pallas import tpu as pltpu
from jax.experimental.pallas import tpu_sc as plsc

UN = 16
FEAT = 512
BATCH = 16
EN = UN * UN * UN
GROUPS = BATCH * UN * UN
LANES = 16
NWORK = 32
GPW = GROUPS // NWORK
W = 2
STEPS = GPW // W
NBUF = 8
REP = 2


def _uncap_body(x_hbm, mask_hbm, out_hbm, mvec, inbuf, outbuf, *sems):
    rsem = sems[:NBUF]
    wsem = sems[NBUF:]
    msem = sems[2 * NBUF]
    nc = 2
    cid = lax.axis_index("c")
    sid = lax.axis_index("s")
    wid = sid * nc + cid
    gbase = wid * GPW
    b = gbase // (UN * UN)
    m0 = gbase - b * (UN * UN)
    srcbase = b * EN + m0

    mask_cp = pltpu.make_async_copy(mask_hbm.at[b], mvec, msem)
    mask_cp.start()

    def read(j, k):
        return pltpu.make_async_copy(
            x_hbm.at[pl.ds(srcbase + j * W, W)], inbuf.at[k], rsem[k]
        )

    for k in range(NBUF):
        read(k, k).start()
    mask_cp.wait()

    def sum_body(i, acc):
        return acc + mvec[pl.ds(i * LANES, LANES)]

    acc = lax.fori_loop(0, EN // LANES, sum_body, jnp.zeros((LANES,), jnp.int32))
    s = jnp.int32(0)
    for i in range(LANES):
        s = s + acc[i]
    s = s.astype(jnp.float32)
    ull = jnp.int32(0)
    for u in range(1, LANES + 1):
        ull = ull + (s >= jnp.float32((u - 0.5) ** 3)).astype(jnp.int32)

    zerov = jnp.zeros((LANES,), jnp.float32)

    def write(j, k, w, t):
        g = gbase + j * W + w
        return pltpu.make_async_copy(
            outbuf.at[k, w], out_hbm.at[pl.ds(g * UN + t * REP, REP)], wsem[k]
        )

    def outer(jj, carry):
        for k in range(NBUF):
            j = jj * NBUF + k
            read(j, k).wait()

            @pl.when(jj > 0)
            def _wait_prev_write():
                for w in range(W):
                    for t in range(UN // REP):
                        write(j - NBUF, k, w, t).wait()

            for w in range(W):
                m = m0 + j * W + w
                i0 = m // UN
                i1 = m - i0 * UN
                valid = (i0 < ull) & (i1 < ull)

                @pl.when(valid)
                def _rep():
                    def rep_body(c, _):
                        c16 = c * LANES
                        v = inbuf[k, w, pl.ds(c16, LANES)]
                        for rep in range(REP):
                            outbuf[k, w, rep, pl.ds(c16, LANES)] = v
                        return 0

                    lax.fori_loop(0, FEAT // LANES, rep_body, 0, unroll=4)

                @pl.when(jnp.logical_not(valid))
                def _zero():
                    def zero_body(c, _):
                        c16 = c * LANES
                        for rep in range(REP):
                            outbuf[k, w, rep, pl.ds(c16, LANES)] = zerov
                        return 0

                    lax.fori_loop(0, FEAT // LANES, zero_body, 0)

                for t in range(UN // REP):
                    write(j, k, w, t).start()

            @pl.when(j + NBUF < STEPS)
            def _next_read():
                read(j + NBUF, k).start()

        return carry

    lax.fori_loop(0, STEPS // NBUF, outer, 0)

    for k in range(NBUF):
        for w in range(W):
            for t in range(UN // REP):
                write(STEPS - NBUF + k, k, w, t).wait()


def kernel(x, mask):
    x2 = x.reshape(BATCH * EN, FEAT)
    mesh = plsc.VectorSubcoreMesh(core_axis_name="c", subcore_axis_name="s")
    run = pl.kernel(
        _uncap_body,
        out_type=jax.ShapeDtypeStruct((GROUPS * UN, FEAT), jnp.float32),
        mesh=mesh,
        scratch_types=(
            [
                pltpu.VMEM((EN,), jnp.int32),
                pltpu.VMEM((NBUF, W, FEAT), jnp.float32),
                pltpu.VMEM((NBUF, W, REP, FEAT), jnp.float32),
            ]
            + [pltpu.SemaphoreType.DMA] * (2 * NBUF + 1)
        ),
    )
    out = run(x2, mask)
    return out.reshape(BATCH, UN, UN, UN, FEAT)
